# Initial kernel scaffold; baseline (speedup 1.0000x reference)
import jax
import jax.numpy as jnp
from jax.experimental import pallas as pl


def kernel(mean_map, variance_map, confidence_map, point_cloud):
    raise NotImplementedError("write your pallas kernel here")


# speedup vs baseline: 1.2866x; 1.2866x over previous
import functools

import jax
import jax.numpy as jnp
import numpy as np
from jax.experimental import pallas as pl
from jax.experimental.pallas import tpu as pltpu

_GRID = (128, 128, 32)
_MIN_B = np.array([-25.6, -25.6, -2.0], dtype=np.float32)
_MAX_B = np.array([25.6, 25.6, 1.2], dtype=np.float32)
_VOX = (_MAX_B - _MIN_B) / np.array(_GRID, dtype=np.float32)
_LATENT = 32
_ELL = 0.5
_SIGMA = 1.0
_N_PTS = 65536
_PI = float(np.pi)

_CHUNK = 128
_N_CHUNKS = _N_PTS // _CHUNK
_ROWS = _GRID[0] * _GRID[2]
_F = _LATENT + 1


def _tap_weight(dx, dy, dz):
    d = float(np.sqrt((dx * _VOX[0]) ** 2 + (dy * _VOX[1]) ** 2 + (dz * _VOX[2]) ** 2))
    if d >= _ELL:
        return 0.0
    kv = _SIGMA * ((1.0 / 3.0) * (2.0 + np.cos(2.0 * _PI * d / _ELL)) * (1.0 - d / _ELL)
                   + (1.0 / (2.0 * _PI)) * np.sin(2.0 * _PI * d / _ELL))
    return float(np.clip(kv, 0.0, 1.0))


_TAPS = [(dx, dy, dz, _tap_weight(dx, dy, dz))
         for dx in (-1, 0, 1) for dy in (-1, 0, 1) for dz in (-1, 0, 1)
         if _tap_weight(dx, dy, dz) != 0.0]


_HALF = _ROWS // 2


def _bin_kernel(half, rows_ref, iys_ref, pft_ref, acc_ref):
    @pl.when(pl.program_id(0) == 0)
    def _():
        acc_ref[...] = jnp.zeros_like(acc_ref)

    base = half * _HALF
    blk = pft_ref[...]
    lane = jax.lax.broadcasted_iota(jnp.int32, (_F, _CHUNK), 1)

    def body(p, carry):
        lrow = rows_ref[0, 0, p] - base
        iy = iys_ref[0, 0, p]
        in_half = (lrow >= 0) & (lrow < _HALF)
        crow = jnp.clip(lrow, 0, _HALF - 1)
        c = jnp.sum(jnp.where(lane == p, blk, 0.0), axis=1, keepdims=True)
        upd = jnp.where(lane == iy, c, 0.0) * jnp.where(in_half, 1.0, 0.0)
        acc_ref[pl.ds(crow, 1)] = acc_ref[pl.ds(crow, 1)] + upd[None]
        return carry

    jax.lax.fori_loop(0, _CHUNK, body, 0)


def _update_kernel(accm_ref, acc0_ref, accp_ref, mean_ref, var_ref, conf_ref,
                   omean_ref, ovar_ref, oconf_ref):
    ix = pl.program_id(0)
    y = jnp.zeros((32, _F, 128), jnp.float32)
    for dx, aref in ((-1, accm_ref), (0, acc0_ref), (1, accp_ref)):
        xx = ix + dx
        m = jnp.where((xx >= 0) & (xx <= _GRID[0] - 1), 1.0, 0.0)
        t = aref[...]
        zrow = jnp.zeros((1, _F, 128), jnp.float32)
        zlane = jnp.zeros((32, _F, 1), jnp.float32)
        for dz in (-1, 0, 1):
            if dz == 0:
                tz = t
            elif dz == 1:
                tz = jnp.concatenate([t[1:], zrow], axis=0)
            else:
                tz = jnp.concatenate([zrow, t[:-1]], axis=0)
            for dy in (-1, 0, 1):
                w = _tap_weight(dx, dy, dz)
                if w == 0.0:
                    continue
                if dy == 0:
                    tzy = tz
                elif dy == 1:
                    tzy = jnp.concatenate([tz[:, :, 1:], zlane], axis=2)
                else:
                    tzy = jnp.concatenate([zlane, tz[:, :, :-1]], axis=2)
                y = y + (w * m) * tzy

    k_bar = y[:, 0:1, :]
    y_bar = y[:, 1:, :]
    conf = conf_ref[...]
    mean = mean_ref[...]
    var = var_ref[...]
    new_conf = conf + k_bar
    denom = jnp.where(new_conf > 0, new_conf, 1.0)
    omean_ref[...] = jnp.where(new_conf > 0, (conf * mean + y_bar) / denom, mean)
    ovar_ref[...] = jnp.where(k_bar > 0, (conf * var + k_bar) / denom, var)
    oconf_ref[...] = new_conf


@jax.jit
def kernel(mean_map, variance_map, confidence_map, point_cloud):
    gx, gy, gz = _GRID
    vox = jnp.asarray(_VOX)
    minb = jnp.asarray(_MIN_B)
    xyz = point_cloud[:, :3]
    feats = point_cloud[:, 3:]

    inds = jnp.floor((xyz - minb) / vox).astype(jnp.int32)
    rows = (inds[:, 0] * gz + inds[:, 2]).reshape(_N_CHUNKS, 1, _CHUNK)
    iys = inds[:, 1].reshape(_N_CHUNKS, 1, _CHUNK)
    pf = jnp.concatenate([jnp.ones((_N_PTS, 1), jnp.float32), feats], axis=1)
    pft = pf.T

    halves = []
    for h in (0, 1):
        halves.append(pl.pallas_call(
            functools.partial(_bin_kernel, h),
            grid=(_N_CHUNKS,),
            in_specs=[
                pl.BlockSpec((1, 1, _CHUNK), lambda i: (i, 0, 0), memory_space=pltpu.SMEM),
                pl.BlockSpec((1, 1, _CHUNK), lambda i: (i, 0, 0), memory_space=pltpu.SMEM),
                pl.BlockSpec((_F, _CHUNK), lambda i: (0, i)),
            ],
            out_specs=pl.BlockSpec((_HALF, _F, 128), lambda i: (0, 0, 0)),
            out_shape=jax.ShapeDtypeStruct((_HALF, _F, 128), jnp.float32),
        )(rows, iys, pft))
    acc = jnp.concatenate(halves, axis=0)

    mean8 = mean_map.transpose(0, 2, 3, 1).reshape(_ROWS, _LATENT, gy)
    var8 = variance_map.transpose(0, 2, 3, 1).reshape(_ROWS, _LATENT, gy)
    conf8 = confidence_map.transpose(0, 2, 3, 1).reshape(_ROWS, 1, gy)

    omean, ovar, oconf = pl.pallas_call(
        _update_kernel,
        grid=(gx,),
        in_specs=[
            pl.BlockSpec((gz, _F, 128), lambda i: (jnp.maximum(i - 1, 0), 0, 0)),
            pl.BlockSpec((gz, _F, 128), lambda i: (i, 0, 0)),
            pl.BlockSpec((gz, _F, 128), lambda i: (jnp.minimum(i + 1, gx - 1), 0, 0)),
            pl.BlockSpec((gz, _LATENT, gy), lambda i: (i, 0, 0)),
            pl.BlockSpec((gz, _LATENT, gy), lambda i: (i, 0, 0)),
            pl.BlockSpec((gz, 1, gy), lambda i: (i, 0, 0)),
        ],
        out_specs=[
            pl.BlockSpec((gz, _LATENT, gy), lambda i: (i, 0, 0)),
            pl.BlockSpec((gz, _LATENT, gy), lambda i: (i, 0, 0)),
            pl.BlockSpec((gz, 1, gy), lambda i: (i, 0, 0)),
        ],
        out_shape=[
            jax.ShapeDtypeStruct((_ROWS, _LATENT, gy), jnp.float32),
            jax.ShapeDtypeStruct((_ROWS, _LATENT, gy), jnp.float32),
            jax.ShapeDtypeStruct((_ROWS, 1, gy), jnp.float32),
        ],
    )(acc, acc, acc, mean8, var8, conf8)

    new_mean = omean.reshape(gx, gz, _LATENT, gy).transpose(0, 3, 1, 2)
    new_var = ovar.reshape(gx, gz, _LATENT, gy).transpose(0, 3, 1, 2)
    new_conf = oconf.reshape(gx, gz, 1, gy).transpose(0, 3, 1, 2)
    return new_mean, new_var, new_conf
